# Initial kernel scaffold; baseline (speedup 1.0000x reference)
import jax
import jax.numpy as jnp
from jax.experimental import pallas as pl


def kernel(idx0, idx1, table0, table1):
    raise NotImplementedError("write your pallas kernel here")


# speedup vs baseline: 1.6163x; 1.6163x over previous
import functools

import jax
import jax.numpy as jnp
from jax import lax
from jax.experimental import pallas as pl
from jax.experimental.pallas import tpu as pltpu
from jax.experimental.pallas import tpu_sc as plsc

VOCAB = 1000000
DIM = 32
B = 16384
L = 20
BL = B * L

_info = plsc.get_sparse_core_info()
NC, NS = _info.num_cores, _info.num_subcores
NW = NC * NS
PW = BL // NW
NCH = 5
CH = PW // NCH


def _body(idx0_hbm, idx1_hbm, tab0_hbm, tab1_hbm, out_hbm, idx_v, rows_v, sem):
    wid = lax.axis_index("s") * NC + lax.axis_index("c")
    base = wid * PW
    for t, (idx_hbm, tab_hbm) in enumerate(((idx0_hbm, tab0_hbm),
                                            (idx1_hbm, tab1_hbm))):
        for c in range(NCH):
            off = base + c * CH
            pltpu.sync_copy(idx_hbm.at[pl.ds(off, CH)], idx_v)
            pltpu.async_copy(tab_hbm.at[idx_v], rows_v, sem).wait()
            pltpu.sync_copy(rows_v, out_hbm.at[pl.ds(t * BL + off, CH)])


_mesh = plsc.VectorSubcoreMesh(core_axis_name="c", subcore_axis_name="s")

_sc_gather = pl.kernel(
    _body,
    out_type=jax.ShapeDtypeStruct((2 * BL, DIM), jnp.float32),
    mesh=_mesh,
    scratch_types=[
        pltpu.VMEM((CH,), jnp.int32),
        pltpu.VMEM((CH, DIM), jnp.float32),
        pltpu.SemaphoreType.DMA,
    ],
    compiler_params=pltpu.CompilerParams(use_tc_tiling_on_sc=False),
)


@jax.jit
def kernel(idx0, idx1, table0, table1):
    i0 = idx0.reshape(BL).astype(jnp.int32)
    i1 = idx1.reshape(BL).astype(jnp.int32)
    out = _sc_gather(i0, i1, table0, table1)
    return out.reshape(2, B, L, DIM)


# speedup vs baseline: 1.6220x; 1.0035x over previous
import jax
import jax.numpy as jnp
from jax import lax
from jax.experimental import pallas as pl
from jax.experimental.pallas import tpu as pltpu
from jax.experimental.pallas import tpu_sc as plsc

VOCAB = 1000000
DIM = 32
B = 16384
L = 20
BL = B * L

_info = plsc.get_sparse_core_info()
NC, NS = _info.num_cores, _info.num_subcores
NW = NC * NS
PW = BL // NW
NCH = 8
CH = PW // NCH


def _body(idx0_hbm, idx1_hbm, tab0_hbm, tab1_hbm, out_hbm,
          idx_v0, idx_v1, rows_v0, rows_v1,
          sem_i0, sem_i1, sem_g0, sem_g1, sem_o0, sem_o1):
    wid = lax.axis_index("s") * NC + lax.axis_index("c")
    base = wid * PW

    idx_bufs = (idx_v0, idx_v1)
    rows_bufs = (rows_v0, rows_v1)
    sems_i = (sem_i0, sem_i1)
    sems_g = (sem_g0, sem_g1)
    sems_o = (sem_o0, sem_o1)

    chunks = []
    for t, (idx_hbm, tab_hbm) in enumerate(((idx0_hbm, tab0_hbm),
                                            (idx1_hbm, tab1_hbm))):
        for c in range(NCH):
            chunks.append((idx_hbm, tab_hbm, base + c * CH, t * BL))
    T = len(chunks)

    def start_idx(c):
        idx_hbm, _, off, _ = chunks[c]
        b = c % 2
        return pltpu.async_copy(idx_hbm.at[pl.ds(off, CH)], idx_bufs[b],
                                sems_i[b])

    def start_gather(c):
        _, tab_hbm, _, _ = chunks[c]
        b = c % 2
        return pltpu.async_copy(tab_hbm.at[idx_bufs[b]], rows_bufs[b],
                                sems_g[b])

    def start_out(c):
        _, _, off, obase = chunks[c]
        b = c % 2
        return pltpu.async_copy(rows_bufs[b],
                                out_hbm.at[pl.ds(obase + off, CH)],
                                sems_o[b])

    idx_h = [None] * T
    gat_h = [None] * T
    out_h = [None] * T

    idx_h[0] = start_idx(0)
    idx_h[1] = start_idx(1)
    for c in range(T):
        idx_h[c].wait()
        if c >= 2:
            out_h[c - 2].wait()
        gat_h[c] = start_gather(c)
        if c >= 1:
            gat_h[c - 1].wait()
            out_h[c - 1] = start_out(c - 1)
            if c + 1 < T:
                idx_h[c + 1] = start_idx(c + 1)
    gat_h[T - 1].wait()
    out_h[T - 1] = start_out(T - 1)
    out_h[T - 2].wait()
    out_h[T - 1].wait()


_mesh = plsc.VectorSubcoreMesh(core_axis_name="c", subcore_axis_name="s")

_sc_gather = pl.kernel(
    _body,
    out_type=jax.ShapeDtypeStruct((2 * BL, DIM), jnp.float32),
    mesh=_mesh,
    scratch_types=[
        pltpu.VMEM((CH,), jnp.int32),
        pltpu.VMEM((CH,), jnp.int32),
        pltpu.VMEM((CH, DIM), jnp.float32),
        pltpu.VMEM((CH, DIM), jnp.float32),
        pltpu.SemaphoreType.DMA,
        pltpu.SemaphoreType.DMA,
        pltpu.SemaphoreType.DMA,
        pltpu.SemaphoreType.DMA,
        pltpu.SemaphoreType.DMA,
        pltpu.SemaphoreType.DMA,
    ],
    compiler_params=pltpu.CompilerParams(use_tc_tiling_on_sc=False),
)


@jax.jit
def kernel(idx0, idx1, table0, table1):
    i0 = idx0.reshape(BL).astype(jnp.int32)
    i1 = idx1.reshape(BL).astype(jnp.int32)
    out = _sc_gather(i0, i1, table0, table1)
    return out.reshape(2, B, L, DIM)


# speedup vs baseline: 1.7378x; 1.0713x over previous
import jax
import jax.numpy as jnp
from jax import lax
from jax.experimental import pallas as pl
from jax.experimental.pallas import tpu as pltpu
from jax.experimental.pallas import tpu_sc as plsc

VOCAB = 1000000
DIM = 32
B = 16384
L = 20
BL = B * L

_info = plsc.get_sparse_core_info()
NC, NS = _info.num_cores, _info.num_subcores
NW = NC * NS
PW = BL // NW
NCH = 8
CH = PW // NCH


def _body(idx0_hbm, idx1_hbm, tab0_hbm, tab1_hbm, out_hbm,
          idx_v0, idx_v1, rows_v0, rows_v1,
          sem_i0, sem_i1, sem_g0, sem_g1, sem_o0, sem_o1):
    wid = lax.axis_index("s") * NC + lax.axis_index("c")
    base = wid * PW

    idx_bufs = (idx_v0, idx_v1)
    rows_bufs = (rows_v0, rows_v1)
    sems_i = (sem_i0, sem_i1)
    sems_g = (sem_g0, sem_g1)
    sems_o = (sem_o0, sem_o1)

    chunks = []
    for t, (idx_hbm, tab_hbm) in enumerate(((idx0_hbm, tab0_hbm),
                                            (idx1_hbm, tab1_hbm))):
        for c in range(NCH):
            chunks.append((idx_hbm, tab_hbm, base + c * CH, t * BL))
    T = len(chunks)

    def start_idx(c):
        idx_hbm, _, off, _ = chunks[c]
        b = c % 2
        return pltpu.async_copy(idx_hbm.at[pl.ds(off, CH)], idx_bufs[b],
                                sems_i[b])

    def start_gather(c):
        _, tab_hbm, _, _ = chunks[c]
        b = c % 2
        return pltpu.async_copy(tab_hbm.at[idx_bufs[b]], rows_bufs[b],
                                sems_g[b])

    def start_out(c):
        _, _, off, obase = chunks[c]
        b = c % 2
        return pltpu.async_copy(rows_bufs[b],
                                out_hbm.at[pl.ds(obase + off, CH)],
                                sems_o[b])

    idx_h = [None] * T
    gat_h = [None] * T
    out_h = [None] * T

    idx_h[0] = start_idx(0)
    idx_h[1] = start_idx(1)
    for c in range(T):
        idx_h[c].wait()
        if c >= 2:
            out_h[c - 2].wait()
        gat_h[c] = start_gather(c)
        if c >= 1:
            gat_h[c - 1].wait()
            out_h[c - 1] = start_out(c - 1)
            if c + 1 < T:
                idx_h[c + 1] = start_idx(c + 1)
    gat_h[T - 1].wait()
    out_h[T - 1] = start_out(T - 1)
    out_h[T - 2].wait()
    out_h[T - 1].wait()


_mesh = plsc.VectorSubcoreMesh(core_axis_name="c", subcore_axis_name="s")

_sc_gather = pl.kernel(
    _body,
    out_type=jax.ShapeDtypeStruct((2 * BL, DIM), jnp.float32),
    mesh=_mesh,
    scratch_types=[
        pltpu.VMEM((CH,), jnp.int32),
        pltpu.VMEM((CH,), jnp.int32),
        pltpu.VMEM((CH, DIM), jnp.float32),
        pltpu.VMEM((CH, DIM), jnp.float32),
        pltpu.SemaphoreType.DMA,
        pltpu.SemaphoreType.DMA,
        pltpu.SemaphoreType.DMA,
        pltpu.SemaphoreType.DMA,
        pltpu.SemaphoreType.DMA,
        pltpu.SemaphoreType.DMA,
    ],
    compiler_params=pltpu.CompilerParams(use_tc_tiling_on_sc=False),
)


@jax.jit
def kernel(idx0, idx1, table0, table1):
    i0 = idx0.T.reshape(BL).astype(jnp.int32)
    i1 = idx1.T.reshape(BL).astype(jnp.int32)
    out = _sc_gather(i0, i1, table0, table1)
    return out.reshape(2, L, B, DIM).transpose(0, 2, 1, 3)
